# baseline (speedup 1.0000x reference)
import jax
import jax.numpy as jnp
from jax import lax
from jax.experimental import pallas as pl
from jax.experimental.pallas import tpu as pltpu
from jax.experimental.pallas import tpu_sc as plsc

_N = 10000
_D = 128
_H = 64
_E = 320000
_ITERS = 4

_NC = 2
_NS = 16
_NW = _NC * _NS
_CH = 128
_GCH = 512
_SUB = _GCH // _CH
_EPT = -(-_E // _NW)
_GCHUNKS = -(-_EPT // _GCH)
_CHUNKS = _GCHUNKS * _SUB
_E_PAD = _CHUNKS * _CH * _NW
_NPAD = 10112
_RPT = _NPAD // _NS


def _sc_agg_body(m_hbm, src_hbm, dst_hbm, out0_hbm, out1_hbm,
                 idx_s, idx_d, rows0, rows1, acc, g0, g1):
    c = lax.axis_index("c")
    s = lax.axis_index("s")
    wid = s * _NC + c
    row0 = s * _RPT
    rem = _RPT - _GCH

    pltpu.sync_copy(src_hbm.at[wid], idx_s)
    pltpu.sync_copy(dst_hbm.at[wid], idx_d)

    zv = jnp.zeros((16,), jnp.float32)

    def _zero_row(i, carry):
        for j in range(_H // 16):
            rows0[i, pl.ds(j * 16, 16)] = zv
        return carry

    lax.fori_loop(0, _GCH, _zero_row, 0)
    pltpu.sync_copy(rows0, acc.at[pl.ds(row0, _GCH)])
    pltpu.sync_copy(rows0.at[pl.ds(0, rem)], acc.at[pl.ds(row0 + _GCH, rem)])
    plsc.subcore_barrier()

    def _gather(q, buf, sem):
        pltpu.async_copy(m_hbm.at[idx_s.at[pl.ds(q * _GCH, _GCH)]], buf, sem)

    def _gwait(q, buf, sem):
        pltpu.make_async_copy(
            m_hbm.at[idx_s.at[pl.ds(q * _GCH, _GCH)]], buf, sem).wait()

    def _scatter(q, buf):
        for k in range(_SUB):
            pltpu.sync_copy(buf.at[pl.ds(k * _CH, _CH)],
                            acc.at[idx_d.at[q * _SUB + k]], add=True)

    _gather(0, rows0, g0)

    def _pair(p, carry):
        j0 = 2 * p
        j1 = j0 + 1
        _gather(j1, rows1, g1)
        _gwait(j0, rows0, g0)
        _scatter(j0, rows0)

        @pl.when(j1 + 1 < _GCHUNKS)
        def _():
            _gather(j1 + 1, rows0, g0)

        _gwait(j1, rows1, g1)
        _scatter(j1, rows1)
        return carry

    lax.fori_loop(0, _GCHUNKS // 2, _pair, 0)
    plsc.subcore_barrier()

    pltpu.sync_copy(acc.at[pl.ds(row0, _GCH)], rows0)
    pltpu.sync_copy(acc.at[pl.ds(row0 + _GCH, rem)], rows1.at[pl.ds(0, rem)])

    @pl.when(c == 0)
    def _():
        pltpu.sync_copy(rows0, out0_hbm.at[pl.ds(row0, _GCH)])
        pltpu.sync_copy(rows1.at[pl.ds(0, rem)],
                        out0_hbm.at[pl.ds(row0 + _GCH, rem)])

    @pl.when(c == 1)
    def _():
        pltpu.sync_copy(rows0, out1_hbm.at[pl.ds(row0, _GCH)])
        pltpu.sync_copy(rows1.at[pl.ds(0, rem)],
                        out1_hbm.at[pl.ds(row0 + _GCH, rem)])


_sc_agg_cache = []


def _sc_agg(m, srcp, dstp):
    if not _sc_agg_cache:
        _sc_agg_cache.append(pl.kernel(
            _sc_agg_body,
            out_type=(jax.ShapeDtypeStruct((_NPAD, _H), jnp.float32),
                      jax.ShapeDtypeStruct((_NPAD, _H), jnp.float32)),
            mesh=plsc.VectorSubcoreMesh(core_axis_name="c",
                                        subcore_axis_name="s"),
            scratch_types=[
                pltpu.VMEM((_GCHUNKS * _GCH,), jnp.int32),
                pltpu.VMEM((_CHUNKS, _CH), jnp.int32),
                pltpu.VMEM((_GCH, _H), jnp.float32),
                pltpu.VMEM((_GCH, _H), jnp.float32),
                pltpu.VMEM_SHARED((_NPAD, _H), jnp.float32),
                pltpu.SemaphoreType.DMA,
                pltpu.SemaphoreType.DMA,
            ],
            compiler_params=pltpu.CompilerParams(use_tc_tiling_on_sc=False),
        ))
    return _sc_agg_cache[0](m, srcp, dstp)


def _enc_body(x_ref, wi_ref, bi_ref, wm_ref, bm_ref, h_ref, m_ref):
    h = jnp.maximum(
        jnp.dot(x_ref[...], wi_ref[...], preferred_element_type=jnp.float32)
        + bi_ref[...], 0.0)
    h_ref[...] = h
    m_ref[...] = jnp.maximum(
        jnp.dot(h, wm_ref[...], preferred_element_type=jnp.float32)
        + bm_ref[...], 0.0)


def _upd_body(h_ref, a0_ref, a1_ref, wt_ref, wb_ref, bu_ref, wm_ref, bm_ref,
              h_out, m_out):
    agg = a0_ref[...] + a1_ref[...]
    hn = jnp.maximum(
        jnp.dot(h_ref[...], wt_ref[...], preferred_element_type=jnp.float32)
        + jnp.dot(agg, wb_ref[...], preferred_element_type=jnp.float32)
        + bu_ref[...], 0.0)
    h_out[...] = hn
    m_out[...] = jnp.maximum(
        jnp.dot(hn, wm_ref[...], preferred_element_type=jnp.float32)
        + bm_ref[...], 0.0)


def _fin_body(h_ref, a0_ref, a1_ref, wt_ref, wb_ref, bu_ref, wo_ref, bo_ref,
              out_ref):
    agg = a0_ref[...] + a1_ref[...]
    hn = jnp.maximum(
        jnp.dot(h_ref[...], wt_ref[...], preferred_element_type=jnp.float32)
        + jnp.dot(agg, wb_ref[...], preferred_element_type=jnp.float32)
        + bu_ref[...], 0.0)
    out_ref[...] = (
        jnp.dot(hn, wo_ref[...], preferred_element_type=jnp.float32)
        + bo_ref[...])


def _hm_shapes():
    return (jax.ShapeDtypeStruct((_NPAD, _H), jnp.float32),
            jax.ShapeDtypeStruct((_NPAD, _H), jnp.float32))


def kernel(x, edge_index, W_in, b_in, W_msg, b_msg, W_upd, b_upd, W_out, b_out):
    f32 = jnp.float32
    xp = jnp.zeros((_NPAD, _D), f32).at[:_N, :].set(x)
    src = edge_index[0]
    dst = edge_index[1]
    srcp = jnp.concatenate(
        [src, jnp.zeros((_E_PAD - _E,), jnp.int32)]).reshape(
            _NW, _GCHUNKS * _GCH)
    dummy_dst = _N + jnp.arange(_E_PAD - _E, dtype=jnp.int32) % (_NPAD - _N)
    dstp = jnp.concatenate([dst, dummy_dst]).reshape(_NW, _CHUNKS, _CH)
    bi = b_in.reshape(1, _H)
    bm = b_msg.reshape(1, _H)
    bu = b_upd.reshape(1, _H)
    bo = b_out.reshape(1, _H)
    wt = W_upd[:_H]
    wb = W_upd[_H:]

    h, m = pl.pallas_call(_enc_body, out_shape=_hm_shapes())(
        xp, W_in, bi, W_msg, bm)
    out = None
    for it in range(_ITERS):
        a0, a1 = _sc_agg(m, srcp, dstp)
        if it < _ITERS - 1:
            h, m = pl.pallas_call(_upd_body, out_shape=_hm_shapes())(
                h, a0, a1, wt, wb, bu, W_msg, bm)
        else:
            out = pl.pallas_call(
                _fin_body,
                out_shape=jax.ShapeDtypeStruct((_NPAD, _H), f32))(
                    h, a0, a1, wt, wb, bu, W_out, bo)
    return out[:_N]


# speedup vs baseline: 1.3565x; 1.3565x over previous
import jax
import jax.numpy as jnp
from jax import lax
from jax.experimental import pallas as pl
from jax.experimental.pallas import tpu as pltpu
from jax.experimental.pallas import tpu_sc as plsc

_N = 10000
_D = 128
_H = 64
_E = 320000
_ITERS = 4

_NC = 2
_NS = 16
_NW = _NC * _NS
_CH = 128
_CA = 54
_CB = 104
_CMAX = max(_CA, _CB)
_E_PAD = _NS * (_CA + _CB) * _CH
_NPAD = 10112
_RPT = _NPAD // _NS


def _sc_agg_body(m_hbm, src_hbm, dst_hbm, out0_hbm, out1_hbm,
                 idx_s, idx_d, rows0, rows1, zblk, acc, g0, g1):
    c = lax.axis_index("c")
    s = lax.axis_index("s")
    wid = s * _NC + c
    row0 = s * _RPT
    nch = jnp.where(c == 0, _CA, _CB)

    pltpu.sync_copy(src_hbm.at[wid], idx_s)
    pltpu.sync_copy(dst_hbm.at[wid], idx_d)

    zv = jnp.zeros((16,), jnp.float32)

    def _zero_row(i, carry):
        for j in range(_H // 16):
            zblk[i, pl.ds(j * 16, 16)] = zv
        return carry

    lax.fori_loop(0, _RPT, _zero_row, 0)
    pltpu.sync_copy(zblk, acc.at[pl.ds(row0, _RPT)])
    plsc.subcore_barrier()

    pltpu.async_copy(m_hbm.at[idx_s.at[0]], rows0, g0)

    def _pair(p, carry):
        j0 = 2 * p
        j1 = j0 + 1
        pltpu.async_copy(m_hbm.at[idx_s.at[j1]], rows1, g1)
        pltpu.make_async_copy(m_hbm.at[idx_s.at[j0]], rows0, g0).wait()
        pltpu.sync_copy(rows0, acc.at[idx_d.at[j0]], add=True)

        @pl.when(j1 + 1 < nch)
        def _():
            pltpu.async_copy(m_hbm.at[idx_s.at[j1 + 1]], rows0, g0)

        pltpu.make_async_copy(m_hbm.at[idx_s.at[j1]], rows1, g1).wait()
        pltpu.sync_copy(rows1, acc.at[idx_d.at[j1]], add=True)
        return carry

    lax.fori_loop(0, nch // 2, _pair, 0)
    plsc.subcore_barrier()

    pltpu.sync_copy(acc.at[pl.ds(row0, _RPT)], zblk)

    @pl.when(c == 0)
    def _():
        pltpu.sync_copy(zblk, out0_hbm.at[pl.ds(row0, _RPT)])

    @pl.when(c == 1)
    def _():
        pltpu.sync_copy(zblk, out1_hbm.at[pl.ds(row0, _RPT)])


_sc_agg_cache = []


def _sc_agg(m, srcp, dstp):
    if not _sc_agg_cache:
        _sc_agg_cache.append(pl.kernel(
            _sc_agg_body,
            out_type=(jax.ShapeDtypeStruct((_NPAD, _H), jnp.float32),
                      jax.ShapeDtypeStruct((_NPAD, _H), jnp.float32)),
            mesh=plsc.VectorSubcoreMesh(core_axis_name="c",
                                        subcore_axis_name="s"),
            scratch_types=[
                pltpu.VMEM((_CMAX, _CH), jnp.int32),
                pltpu.VMEM((_CMAX, _CH), jnp.int32),
                pltpu.VMEM((_CH, _H), jnp.float32),
                pltpu.VMEM((_CH, _H), jnp.float32),
                pltpu.VMEM((_RPT, _H), jnp.float32),
                pltpu.VMEM_SHARED((_NPAD, _H), jnp.float32),
                pltpu.SemaphoreType.DMA,
                pltpu.SemaphoreType.DMA,
            ],
            compiler_params=pltpu.CompilerParams(use_tc_tiling_on_sc=False),
        ))
    return _sc_agg_cache[0](m, srcp, dstp)


def _enc_body(x_ref, wi_ref, bi_ref, wm_ref, bm_ref, h_ref, m_ref):
    h = jnp.maximum(
        jnp.dot(x_ref[...], wi_ref[...], preferred_element_type=jnp.float32)
        + bi_ref[...], 0.0)
    h_ref[...] = h
    m_ref[...] = jnp.maximum(
        jnp.dot(h, wm_ref[...], preferred_element_type=jnp.float32)
        + bm_ref[...], 0.0)


def _upd_body(h_ref, a0_ref, a1_ref, wt_ref, wb_ref, bu_ref, wm_ref, bm_ref,
              h_out, m_out):
    agg = a0_ref[...] + a1_ref[...]
    hn = jnp.maximum(
        jnp.dot(h_ref[...], wt_ref[...], preferred_element_type=jnp.float32)
        + jnp.dot(agg, wb_ref[...], preferred_element_type=jnp.float32)
        + bu_ref[...], 0.0)
    h_out[...] = hn
    m_out[...] = jnp.maximum(
        jnp.dot(hn, wm_ref[...], preferred_element_type=jnp.float32)
        + bm_ref[...], 0.0)


def _fin_body(h_ref, a0_ref, a1_ref, wt_ref, wb_ref, bu_ref, wo_ref, bo_ref,
              out_ref):
    agg = a0_ref[...] + a1_ref[...]
    hn = jnp.maximum(
        jnp.dot(h_ref[...], wt_ref[...], preferred_element_type=jnp.float32)
        + jnp.dot(agg, wb_ref[...], preferred_element_type=jnp.float32)
        + bu_ref[...], 0.0)
    out_ref[...] = (
        jnp.dot(hn, wo_ref[...], preferred_element_type=jnp.float32)
        + bo_ref[...])


def _hm_shapes():
    return (jax.ShapeDtypeStruct((_NPAD, _H), jnp.float32),
            jax.ShapeDtypeStruct((_NPAD, _H), jnp.float32))


def kernel(x, edge_index, W_in, b_in, W_msg, b_msg, W_upd, b_upd, W_out, b_out):
    f32 = jnp.float32
    xp = jnp.zeros((_NPAD, _D), f32).at[:_N, :].set(x)
    src = edge_index[0]
    dst = edge_index[1]
    dummy_dst = _N + jnp.arange(_E_PAD - _E, dtype=jnp.int32) % (_NPAD - _N)
    src_flat = jnp.concatenate([src, jnp.zeros((_E_PAD - _E,), jnp.int32)])
    dst_flat = jnp.concatenate([dst, dummy_dst])

    def _partition(flat):
        parts = []
        off = 0
        for w in range(_NW):
            cnt = _CA if (w % _NC) == 0 else _CB
            blk = flat[off:off + cnt * _CH].reshape(cnt, _CH)
            parts.append(jnp.pad(blk, ((0, _CMAX - cnt), (0, 0))))
            off += cnt * _CH
        return jnp.stack(parts)

    srcp = _partition(src_flat)
    dstp = _partition(dst_flat)
    bi = b_in.reshape(1, _H)
    bm = b_msg.reshape(1, _H)
    bu = b_upd.reshape(1, _H)
    bo = b_out.reshape(1, _H)
    wt = W_upd[:_H]
    wb = W_upd[_H:]

    h, m = pl.pallas_call(_enc_body, out_shape=_hm_shapes())(
        xp, W_in, bi, W_msg, bm)
    out = None
    for it in range(_ITERS):
        a0, a1 = _sc_agg(m, srcp, dstp)
        if it < _ITERS - 1:
            h, m = pl.pallas_call(_upd_body, out_shape=_hm_shapes())(
                h, a0, a1, wt, wb, bu, W_msg, bm)
        else:
            out = pl.pallas_call(
                _fin_body,
                out_shape=jax.ShapeDtypeStruct((_NPAD, _H), f32))(
                    h, a0, a1, wt, wb, bu, W_out, bo)
    return out[:_N]


# speedup vs baseline: 1.6175x; 1.1924x over previous
import jax
import jax.numpy as jnp
from jax import lax
from jax.experimental import pallas as pl
from jax.experimental.pallas import tpu as pltpu
from jax.experimental.pallas import tpu_sc as plsc

_N = 10000
_D = 128
_H = 64
_E = 320000
_ITERS = 4

_NC = 2
_NS = 16
_NW = _NC * _NS
_CH = 128
_CA = 104
_CB = 54
_CMAX = max(_CA, _CB)
_E_PAD = _NS * (_CA + _CB) * _CH
_NPAD = 10112
_RPT = _NPAD // _NS


def _sc_agg_body(m_hbm, src_hbm, dst_hbm, out0_hbm, out1_hbm,
                 idx_s, idx_d, rows0, rows1, zblk, acc, g0, g1):
    c = lax.axis_index("c")
    s = lax.axis_index("s")
    wid = s * _NC + c
    row0 = s * _RPT
    nch = jnp.where(c == 0, _CA, _CB)

    pltpu.sync_copy(src_hbm.at[wid], idx_s)
    pltpu.sync_copy(dst_hbm.at[wid], idx_d)

    zv = jnp.zeros((16,), jnp.float32)

    def _zero_row(i, carry):
        for j in range(_H // 16):
            zblk[i, pl.ds(j * 16, 16)] = zv
        return carry

    lax.fori_loop(0, _RPT, _zero_row, 0)
    pltpu.sync_copy(zblk, acc.at[pl.ds(row0, _RPT)])
    plsc.subcore_barrier()

    pltpu.async_copy(m_hbm.at[idx_s.at[0]], rows0, g0)

    def _pair(p, carry):
        j0 = 2 * p
        j1 = j0 + 1
        pltpu.async_copy(m_hbm.at[idx_s.at[j1]], rows1, g1)
        pltpu.make_async_copy(m_hbm.at[idx_s.at[j0]], rows0, g0).wait()
        pltpu.sync_copy(rows0, acc.at[idx_d.at[j0]], add=True)

        @pl.when(j1 + 1 < nch)
        def _():
            pltpu.async_copy(m_hbm.at[idx_s.at[j1 + 1]], rows0, g0)

        pltpu.make_async_copy(m_hbm.at[idx_s.at[j1]], rows1, g1).wait()
        pltpu.sync_copy(rows1, acc.at[idx_d.at[j1]], add=True)
        return carry

    lax.fori_loop(0, nch // 2, _pair, 0)
    plsc.subcore_barrier()

    pltpu.sync_copy(acc.at[pl.ds(row0, _RPT)], zblk)

    @pl.when(c == 0)
    def _():
        pltpu.sync_copy(zblk, out0_hbm.at[pl.ds(row0, _RPT)])

    @pl.when(c == 1)
    def _():
        pltpu.sync_copy(zblk, out1_hbm.at[pl.ds(row0, _RPT)])


_sc_agg_cache = []


def _sc_agg(m, srcp, dstp):
    if not _sc_agg_cache:
        _sc_agg_cache.append(pl.kernel(
            _sc_agg_body,
            out_type=(jax.ShapeDtypeStruct((_NPAD, _H), jnp.float32),
                      jax.ShapeDtypeStruct((_NPAD, _H), jnp.float32)),
            mesh=plsc.VectorSubcoreMesh(core_axis_name="c",
                                        subcore_axis_name="s"),
            scratch_types=[
                pltpu.VMEM((_CMAX, _CH), jnp.int32),
                pltpu.VMEM((_CMAX, _CH), jnp.int32),
                pltpu.VMEM((_CH, _H), jnp.float32),
                pltpu.VMEM((_CH, _H), jnp.float32),
                pltpu.VMEM((_RPT, _H), jnp.float32),
                pltpu.VMEM_SHARED((_NPAD, _H), jnp.float32),
                pltpu.SemaphoreType.DMA,
                pltpu.SemaphoreType.DMA,
            ],
            compiler_params=pltpu.CompilerParams(use_tc_tiling_on_sc=False),
        ))
    return _sc_agg_cache[0](m, srcp, dstp)


def _enc_body(x_ref, wi_ref, bi_ref, wm_ref, bm_ref, h_ref, m_ref):
    h = jnp.maximum(
        jnp.dot(x_ref[...], wi_ref[...], preferred_element_type=jnp.float32)
        + bi_ref[...], 0.0)
    h_ref[...] = h
    m_ref[...] = jnp.maximum(
        jnp.dot(h, wm_ref[...], preferred_element_type=jnp.float32)
        + bm_ref[...], 0.0)


def _upd_body(h_ref, a0_ref, a1_ref, wt_ref, wb_ref, bu_ref, wm_ref, bm_ref,
              h_out, m_out):
    agg = a0_ref[...] + a1_ref[...]
    hn = jnp.maximum(
        jnp.dot(h_ref[...], wt_ref[...], preferred_element_type=jnp.float32)
        + jnp.dot(agg, wb_ref[...], preferred_element_type=jnp.float32)
        + bu_ref[...], 0.0)
    h_out[...] = hn
    m_out[...] = jnp.maximum(
        jnp.dot(hn, wm_ref[...], preferred_element_type=jnp.float32)
        + bm_ref[...], 0.0)


def _fin_body(h_ref, a0_ref, a1_ref, wt_ref, wb_ref, bu_ref, wo_ref, bo_ref,
              out_ref):
    agg = a0_ref[...] + a1_ref[...]
    hn = jnp.maximum(
        jnp.dot(h_ref[...], wt_ref[...], preferred_element_type=jnp.float32)
        + jnp.dot(agg, wb_ref[...], preferred_element_type=jnp.float32)
        + bu_ref[...], 0.0)
    out_ref[...] = (
        jnp.dot(hn, wo_ref[...], preferred_element_type=jnp.float32)
        + bo_ref[...])


def _hm_shapes():
    return (jax.ShapeDtypeStruct((_NPAD, _H), jnp.float32),
            jax.ShapeDtypeStruct((_NPAD, _H), jnp.float32))


def kernel(x, edge_index, W_in, b_in, W_msg, b_msg, W_upd, b_upd, W_out, b_out):
    f32 = jnp.float32
    xp = jnp.zeros((_NPAD, _D), f32).at[:_N, :].set(x)
    src = edge_index[0]
    dst = edge_index[1]
    dummy_dst = _N + jnp.arange(_E_PAD - _E, dtype=jnp.int32) % (_NPAD - _N)
    src_flat = jnp.concatenate([src, jnp.zeros((_E_PAD - _E,), jnp.int32)])
    dst_flat = jnp.concatenate([dst, dummy_dst])

    def _partition(flat):
        parts = []
        off = 0
        for w in range(_NW):
            cnt = _CA if (w % _NC) == 0 else _CB
            blk = flat[off:off + cnt * _CH].reshape(cnt, _CH)
            parts.append(jnp.pad(blk, ((0, _CMAX - cnt), (0, 0))))
            off += cnt * _CH
        return jnp.stack(parts)

    srcp = _partition(src_flat)
    dstp = _partition(dst_flat)
    bi = b_in.reshape(1, _H)
    bm = b_msg.reshape(1, _H)
    bu = b_upd.reshape(1, _H)
    bo = b_out.reshape(1, _H)
    wt = W_upd[:_H]
    wb = W_upd[_H:]

    h, m = pl.pallas_call(_enc_body, out_shape=_hm_shapes())(
        xp, W_in, bi, W_msg, bm)
    out = None
    for it in range(_ITERS):
        a0, a1 = _sc_agg(m, srcp, dstp)
        if it < _ITERS - 1:
            h, m = pl.pallas_call(_upd_body, out_shape=_hm_shapes())(
                h, a0, a1, wt, wb, bu, W_msg, bm)
        else:
            out = pl.pallas_call(
                _fin_body,
                out_shape=jax.ShapeDtypeStruct((_NPAD, _H), f32))(
                    h, a0, a1, wt, wb, bu, W_out, bo)
    return out[:_N]


# speedup vs baseline: 1.6609x; 1.0268x over previous
import jax
import jax.numpy as jnp
from jax import lax
from jax.experimental import pallas as pl
from jax.experimental.pallas import tpu as pltpu
from jax.experimental.pallas import tpu_sc as plsc

_N = 10000
_D = 128
_H = 64
_E = 320000
_ITERS = 4

_NC = 2
_NS = 16
_NW = _NC * _NS
_CH = 128
_CA = 104
_CB = 54
_CMAX = max(_CA, _CB)
_E_PAD = _NS * (_CA + _CB) * _CH
_NPAD = 10112
_RPT = _NPAD // _NS


def _sc_agg_body(m_hbm, src_hbm, dst_hbm, out0_hbm, out1_hbm,
                 idx_s, idx_d, rows0, rows1, zblk, acc, g0, g1):
    c = lax.axis_index("c")
    s = lax.axis_index("s")
    wid = s * _NC + c
    row0 = s * _RPT
    nch = jnp.where(c == 0, _CA, _CB)

    pltpu.sync_copy(src_hbm.at[wid], idx_s)
    pltpu.sync_copy(dst_hbm.at[wid], idx_d)

    pltpu.async_copy(m_hbm.at[idx_s.at[0]], rows0, g0)

    zv = jnp.zeros((16,), jnp.float32)

    def _zero_row(i, carry):
        for j in range(_H // 16):
            zblk[i, pl.ds(j * 16, 16)] = zv
        return carry

    lax.fori_loop(0, _RPT, _zero_row, 0)
    pltpu.sync_copy(zblk, acc.at[pl.ds(row0, _RPT)])
    plsc.subcore_barrier()


    def _pair(p, carry):
        j0 = 2 * p
        j1 = j0 + 1
        pltpu.async_copy(m_hbm.at[idx_s.at[j1]], rows1, g1)
        pltpu.make_async_copy(m_hbm.at[idx_s.at[j0]], rows0, g0).wait()
        pltpu.sync_copy(rows0, acc.at[idx_d.at[j0]], add=True)

        @pl.when(j1 + 1 < nch)
        def _():
            pltpu.async_copy(m_hbm.at[idx_s.at[j1 + 1]], rows0, g0)

        pltpu.make_async_copy(m_hbm.at[idx_s.at[j1]], rows1, g1).wait()
        pltpu.sync_copy(rows1, acc.at[idx_d.at[j1]], add=True)
        return carry

    lax.fori_loop(0, nch // 2, _pair, 0)
    plsc.subcore_barrier()

    pltpu.sync_copy(acc.at[pl.ds(row0, _RPT)], zblk)

    @pl.when(c == 0)
    def _():
        pltpu.sync_copy(zblk, out0_hbm.at[pl.ds(row0, _RPT)])

    @pl.when(c == 1)
    def _():
        pltpu.sync_copy(zblk, out1_hbm.at[pl.ds(row0, _RPT)])


_sc_agg_cache = []


def _sc_agg(m, srcp, dstp):
    if not _sc_agg_cache:
        _sc_agg_cache.append(pl.kernel(
            _sc_agg_body,
            out_type=(jax.ShapeDtypeStruct((_NPAD, _H), jnp.float32),
                      jax.ShapeDtypeStruct((_NPAD, _H), jnp.float32)),
            mesh=plsc.VectorSubcoreMesh(core_axis_name="c",
                                        subcore_axis_name="s"),
            scratch_types=[
                pltpu.VMEM((_CMAX, _CH), jnp.int32),
                pltpu.VMEM((_CMAX, _CH), jnp.int32),
                pltpu.VMEM((_CH, _H), jnp.float32),
                pltpu.VMEM((_CH, _H), jnp.float32),
                pltpu.VMEM((_RPT, _H), jnp.float32),
                pltpu.VMEM_SHARED((_NPAD, _H), jnp.float32),
                pltpu.SemaphoreType.DMA,
                pltpu.SemaphoreType.DMA,
            ],
            compiler_params=pltpu.CompilerParams(use_tc_tiling_on_sc=False),
        ))
    return _sc_agg_cache[0](m, srcp, dstp)


def _enc_body(x_ref, wi_ref, bi_ref, wm_ref, bm_ref, h_ref, m_ref):
    h = jnp.maximum(
        jnp.dot(x_ref[...], wi_ref[...], preferred_element_type=jnp.float32)
        + bi_ref[...], 0.0)
    h_ref[...] = h
    m_ref[...] = jnp.maximum(
        jnp.dot(h, wm_ref[...], preferred_element_type=jnp.float32)
        + bm_ref[...], 0.0)


def _upd_body(h_ref, a0_ref, a1_ref, wt_ref, wb_ref, bu_ref, wm_ref, bm_ref,
              h_out, m_out):
    agg = a0_ref[...] + a1_ref[...]
    hn = jnp.maximum(
        jnp.dot(h_ref[...], wt_ref[...], preferred_element_type=jnp.float32)
        + jnp.dot(agg, wb_ref[...], preferred_element_type=jnp.float32)
        + bu_ref[...], 0.0)
    h_out[...] = hn
    m_out[...] = jnp.maximum(
        jnp.dot(hn, wm_ref[...], preferred_element_type=jnp.float32)
        + bm_ref[...], 0.0)


def _fin_body(h_ref, a0_ref, a1_ref, wt_ref, wb_ref, bu_ref, wo_ref, bo_ref,
              out_ref):
    agg = a0_ref[...] + a1_ref[...]
    hn = jnp.maximum(
        jnp.dot(h_ref[...], wt_ref[...], preferred_element_type=jnp.float32)
        + jnp.dot(agg, wb_ref[...], preferred_element_type=jnp.float32)
        + bu_ref[...], 0.0)
    out_ref[...] = (
        jnp.dot(hn, wo_ref[...], preferred_element_type=jnp.float32)
        + bo_ref[...])


def _hm_shapes():
    return (jax.ShapeDtypeStruct((_NPAD, _H), jnp.float32),
            jax.ShapeDtypeStruct((_NPAD, _H), jnp.float32))


def kernel(x, edge_index, W_in, b_in, W_msg, b_msg, W_upd, b_upd, W_out, b_out):
    f32 = jnp.float32
    xp = jnp.zeros((_NPAD, _D), f32).at[:_N, :].set(x)
    src = edge_index[0]
    dst = edge_index[1]
    dummy_dst = _N + jnp.arange(_E_PAD - _E, dtype=jnp.int32) % (_NPAD - _N)
    src_flat = jnp.concatenate([src, jnp.zeros((_E_PAD - _E,), jnp.int32)])
    dst_flat = jnp.concatenate([dst, dummy_dst])

    def _partition(flat):
        parts = []
        off = 0
        for w in range(_NW):
            cnt = _CA if (w % _NC) == 0 else _CB
            blk = flat[off:off + cnt * _CH].reshape(cnt, _CH)
            parts.append(jnp.pad(blk, ((0, _CMAX - cnt), (0, 0))))
            off += cnt * _CH
        return jnp.stack(parts)

    srcp = _partition(src_flat)
    dstp = _partition(dst_flat)
    bi = b_in.reshape(1, _H)
    bm = b_msg.reshape(1, _H)
    bu = b_upd.reshape(1, _H)
    bo = b_out.reshape(1, _H)
    wt = W_upd[:_H]
    wb = W_upd[_H:]

    h, m = pl.pallas_call(_enc_body, out_shape=_hm_shapes())(
        xp, W_in, bi, W_msg, bm)
    out = None
    for it in range(_ITERS):
        a0, a1 = _sc_agg(m, srcp, dstp)
        if it < _ITERS - 1:
            h, m = pl.pallas_call(_upd_body, out_shape=_hm_shapes())(
                h, a0, a1, wt, wb, bu, W_msg, bm)
        else:
            out = pl.pallas_call(
                _fin_body,
                out_shape=jax.ShapeDtypeStruct((_NPAD, _H), f32))(
                    h, a0, a1, wt, wb, bu, W_out, bo)
    return out[:_N]
